# baseline (speedup 1.0000x reference)
import jax
import jax.numpy as jnp
from jax import lax
from jax.experimental import pallas as pl
from jax.experimental.pallas import tpu as pltpu
from jax.experimental.pallas import tpu_sc as plsc

DIM = 4096
BATCH = 16384
ROWS = 4

_INFO = plsc.get_sparse_core_info()
_NC = _INFO.num_cores
_NS = _INFO.num_subcores
_NW = _NC * _NS
_BPW = BATCH // _NW
_NTRI = 112
_NSING = 176
_G = 16
_LAG = 3
_SCHED = ["T", "S"] * 7 + ["S", "S"]


def _body(ids_hbm, table_hbm, out_hbm, ids_v, table_v, trip_s, bsem, dsem, ssem):
    cid = lax.axis_index("c")
    sid = lax.axis_index("s")
    wid = sid * _NC + cid
    base = wid * _BPW
    pltpu.sync_copy(ids_hbm.at[pl.ds(base, _BPW)], ids_v)
    pltpu.sync_copy(table_hbm, table_v)

    for k in range(4):
        c = sid * 4 + k
        r0 = c // 16
        r1 = (c // 4) % 4
        r2 = c % 4
        pltpu.async_copy(table_hbm.at[pl.ds(r0 * DIM, DIM)], trip_s.at[pl.ds(c * 3 * DIM, DIM)], bsem)
        pltpu.async_copy(table_hbm.at[pl.ds(r1 * DIM, DIM)], trip_s.at[pl.ds((c * 3 + 1) * DIM, DIM)], bsem)
        pltpu.async_copy(table_hbm.at[pl.ds(r2 * DIM, DIM)], trip_s.at[pl.ds((c * 3 + 2) * DIM, DIM)], bsem)

    def issue_triples(tb):
        i0 = 3 * tb
        vs = (
            ids_v[pl.ds(i0, 16)],
            ids_v[pl.ds(i0 + 16, 16)],
            ids_v[pl.ds(i0 + 32, 16)],
        )
        for j in range(_G):
            e0 = vs[(3 * j) // 16][(3 * j) % 16]
            e1 = vs[(3 * j + 1) // 16][(3 * j + 1) % 16]
            e2 = vs[(3 * j + 2) // 16][(3 * j + 2) % 16]
            combo = e0 * 16 + e1 * 4 + e2
            pltpu.async_copy(
                trip_s.at[pl.ds(combo * (3 * DIM), 3 * DIM)],
                out_hbm.at[pl.ds((base + 3 * (tb + j)) * DIM, 3 * DIM)],
                dsem,
            )

    def issue_singles(sb):
        v = ids_v[pl.ds(3 * _NTRI + sb, 16)]
        for j in range(_G):
            e = v[j]
            pltpu.async_copy(
                table_v.at[pl.ds(e * DIM, DIM)],
                out_hbm.at[pl.ds((base + 3 * _NTRI + sb + j) * DIM, DIM)],
                ssem,
            )

    def drain_triples():
        for _ in range(_G):
            pltpu.make_async_copy(trip_s.at[pl.ds(0, 3 * DIM)], out_hbm.at[pl.ds(0, 3 * DIM)], dsem).wait()

    def drain_singles():
        for _ in range(_G):
            pltpu.make_async_copy(table_v.at[pl.ds(0, DIM)], out_hbm.at[pl.ds(0, DIM)], ssem).wait()

    issue_singles(0)
    issue_singles(16)
    for _ in range(12):
        pltpu.make_async_copy(table_hbm.at[pl.ds(0, DIM)], trip_s.at[pl.ds(0, DIM)], bsem).wait()
    plsc.subcore_barrier()

    t_issued = 0
    s_issued = 2
    t_drained = 0
    s_drained = 0
    for typ in _SCHED:
        if typ == "T":
            if t_issued - t_drained >= _LAG:
                drain_triples()
                t_drained += 1
            issue_triples(t_issued * _G)
            t_issued += 1
        else:
            if s_issued - s_drained >= _LAG:
                drain_singles()
                s_drained += 1
            issue_singles(s_issued * _G)
            s_issued += 1
    while t_drained < t_issued:
        drain_triples()
        t_drained += 1
    while s_drained < s_issued:
        drain_singles()
        s_drained += 1


def kernel(step_ids, step_embeddings):
    ids = step_ids.astype(jnp.int32)
    out = pl.kernel(
        _body,
        out_type=jax.ShapeDtypeStruct((BATCH * DIM,), jnp.float32),
        mesh=plsc.VectorSubcoreMesh(core_axis_name="c", subcore_axis_name="s"),
        scratch_types=[
            pltpu.VMEM((_BPW,), jnp.int32),
            pltpu.VMEM((ROWS * DIM,), jnp.float32),
            pltpu.VMEM_SHARED((64 * 3 * DIM,), jnp.float32),
            pltpu.SemaphoreType.DMA,
            pltpu.SemaphoreType.DMA,
            pltpu.SemaphoreType.DMA,
        ],
    )(ids, step_embeddings.reshape(-1))
    return out.reshape(BATCH, 1, DIM)
